# baseline (speedup 1.0000x reference)
import functools

import jax
import jax.numpy as jnp
from jax import lax
from jax.experimental import pallas as pl
from jax.experimental.pallas import tpu as pltpu
from jax.experimental.pallas import tpu_sc as plsc

_D = 32
_B = 16384
_NC = 2
_NS = 16
_NW = _NC * _NS
_BPW = _B // _NW


@functools.partial(
    pl.kernel,
    mesh=plsc.VectorSubcoreMesh(core_axis_name="c", subcore_axis_name="s"),
    out_type=jax.ShapeDtypeStruct((_B, _D), jnp.float32),
    scratch_types=[
        pltpu.VMEM((_BPW,), jnp.int32),
        pltpu.VMEM((_BPW, _D), jnp.float32),
        pltpu.SemaphoreType.DMA,
    ],
    compiler_params=pltpu.CompilerParams(use_tc_tiling_on_sc=False),
)
def _sc_gather(idx_hbm, table_hbm, out_hbm, idx_v, rows_v, sem):
    wid = lax.axis_index("s") * _NC + lax.axis_index("c")
    base = wid * _BPW
    pltpu.sync_copy(idx_hbm.at[pl.ds(base, _BPW)], idx_v)
    pltpu.async_copy(table_hbm.at[idx_v], rows_v, sem).wait()
    pltpu.sync_copy(rows_v, out_hbm.at[pl.ds(base, _BPW)])


def kernel(movie_title, table):
    idx = movie_title.astype(jnp.int32)
    return _sc_gather(idx, table)


# speedup vs baseline: 4.6597x; 4.6597x over previous
import functools

import jax
import jax.numpy as jnp
from jax import lax
from jax.experimental import pallas as pl
from jax.experimental.pallas import tpu as pltpu
from jax.experimental.pallas import tpu_sc as plsc

_D = 32
_B = 16384
_V = 1000000
_NC = 2
_NS = 16
_L = 16
_NW = _NC * _NS
_IPW = _B // _NW
_NG = _IPW // _L


@functools.partial(
    pl.kernel,
    mesh=plsc.VectorSubcoreMesh(core_axis_name="c", subcore_axis_name="s"),
    out_type=jax.ShapeDtypeStruct((_D, _B), jnp.float32),
    scratch_types=[
        pltpu.VMEM((_IPW,), jnp.int32),
        pltpu.VMEM((_L, _D, 128), jnp.float32),
        pltpu.VMEM((_D, _IPW), jnp.float32),
        pltpu.SemaphoreType.DMA,
        pltpu.SemaphoreType.DMA,
    ],
    compiler_params=pltpu.CompilerParams(needs_layout_passes=False),
)
def _sc_gather(idx_hbm, tabT_hbm, outT_hbm, idx_v, win_v, stage_v, sem, sem2):
    wid = lax.axis_index("s") * _NC + lax.axis_index("c")
    base = wid * _IPW
    cp_idx = pltpu.make_async_copy(idx_hbm.at[pl.ds(base, _IPW)], idx_v, sem2)
    cp_idx.start()
    cp_idx.wait()

    rows0 = lax.iota(jnp.int32, _L)
    rows1 = rows0 + _L

    def issue(v, slot):
        voff = pl.multiple_of((v >> 7) * 128, 128)
        pltpu.make_async_copy(
            tabT_hbm.at[:, pl.ds(voff, 128)], win_v.at[slot], sem
        ).start()

    def wait_one():
        pltpu.make_async_copy(
            tabT_hbm.at[:, pl.ds(0, 128)], win_v.at[0], sem
        ).wait()

    def extract(v, j, slot):
        vin = jnp.full((_L,), v & 127, jnp.int32)
        jcol = jnp.full((_L,), j, jnp.int32)
        v0 = plsc.load_gather(win_v.at[slot], [rows0, vin])
        v1 = plsc.load_gather(win_v.at[slot], [rows1, vin])
        plsc.store_scatter(stage_v, [rows0, jcol], v0)
        plsc.store_scatter(stage_v, [rows1, jcol], v1)

    vec0 = idx_v[pl.ds(0, _L)]
    for k in range(_L):
        issue(vec0[k], k)

    def body(g, prev):
        cur = idx_v[pl.ds(g * _L, _L)]
        for k in range(_L):
            wait_one()
            extract(prev[k], (g - 1) * _L + k, k)
            issue(cur[k], k)
        return cur

    last = lax.fori_loop(1, _NG, body, vec0)
    for k in range(_L):
        wait_one()
        extract(last[k], (_NG - 1) * _L + k, k)

    cp_out = pltpu.make_async_copy(
        stage_v, outT_hbm.at[:, pl.ds(base, _IPW)], sem2
    )
    cp_out.start()
    cp_out.wait()


def kernel(movie_title, table):
    idx = movie_title.astype(jnp.int32)
    outT = _sc_gather(idx, table.T)
    return outT.T
